# baseline (speedup 1.0000x reference)
import jax
import jax.numpy as jnp
from jax.experimental import pallas as pl

FRAMES = 10
_BN_S = float(1.0 / (1.0 + 1e-5) ** 0.5)


def _audio_body(x_ref, w1_ref, w2_ref, w3_ref, sc_ref, bi_ref, fa_ref, ea_ref):
    x = x_ref[0]
    w1 = w1_ref[...]
    sc = sc_ref[...]
    bi = bi_ref[...]
    f32 = jnp.float32

    y0 = jnp.dot(x, w1[0], preferred_element_type=f32)
    y1 = jnp.dot(x, w1[1], preferred_element_type=f32)
    y2 = jnp.dot(x, w1[2], preferred_element_type=f32)
    z8 = jnp.zeros((8, 512), f32)
    y = y1 + jnp.concatenate([z8, y0[:-8]], 0) + jnp.concatenate([y2[8:], z8], 0)
    y = y * sc[0] + bi[0]
    y = y.reshape(100, 2, 4, 512).max(axis=1)
    y = jnp.maximum(y, 0.0)

    p = y.reshape(100, 2, 2, 512)
    ye = p[:, :, 0, :].reshape(200, 512)
    yo = p[:, :, 1, :].reshape(200, 512)
    y = (jnp.dot(ye, w2_ref[0], preferred_element_type=f32)
         + jnp.dot(yo, w2_ref[1], preferred_element_type=f32))
    y = y * sc[1] + bi[1]
    y = jnp.maximum(y, 0.0)

    w3 = w3_ref[...]
    y0 = jnp.dot(y, w3[0], preferred_element_type=f32)
    y1 = jnp.dot(y, w3[1], preferred_element_type=f32)
    y2 = jnp.dot(y, w3[2], preferred_element_type=f32)
    z2 = jnp.zeros((2, 512), f32)
    y = y1 + jnp.concatenate([z2, y0[:-2]], 0) + jnp.concatenate([y2[2:], z2], 0)
    y = y * sc[2] + bi[2]
    y = y.reshape(50, 2, 2, 512).max(axis=1)
    y = jnp.maximum(y, 0.0)

    fa_ref[0] = y.reshape(100, 512)
    ea_ref[0] = y.reshape(10, 5 * 2, 512).max(axis=1)


def _visual_body(x_ref, wv_ref, sc_ref, bi_ref, fv_ref, ev_ref):
    xp = x_ref[0]
    wv = wv_ref[...]
    f32 = jnp.float32
    acc = jnp.zeros((490, 512), f32)
    k = 0
    for dy in range(3):
        for dx in range(3):
            t = xp[:, dy:dy + 7, dx:dx + 7, :].reshape(490, 512)
            acc = acc + jnp.dot(t, wv[k], preferred_element_type=f32)
            k += 1
    y = acc * sc_ref[...] + bi_ref[...]
    y = jnp.maximum(y, 0.0)
    yb = y.reshape(10, 49, 512)
    fv_ref[0] = yb
    ev_ref[0] = yb.max(axis=1)


def _mlp_body(e_ref, w1_ref, b1_ref, w2_ref, b2_ref, o_ref):
    f32 = jnp.float32
    h = jnp.dot(e_ref[...], w1_ref[...], preferred_element_type=f32) + b1_ref[...]
    h = jnp.maximum(h, 0.0)
    o_ref[...] = jnp.dot(h, w2_ref[...], preferred_element_type=f32) + b2_ref[...]


def kernel(audio, visual, W1, g1, b1, W2, g2, b2, W3, g3, b3, Wv, gv, bv, D1w, D1b, D2w, D2b):
    B = audio.shape[0]
    NV = visual.shape[0]
    VB = NV // B

    xa = audio.transpose(0, 2, 3, 1).reshape(B, 800, 512)
    W1t = jnp.transpose(W1[:, :, :, 0], (2, 1, 0))
    W2t = jnp.transpose(W2[:, :, 0, :], (2, 1, 0))
    W3t = jnp.transpose(W3[:, :, :, 0], (2, 1, 0))
    sc_a = jnp.stack([g1, g2, g3]) * _BN_S
    bi_a = jnp.stack([b1, b2, b3])

    xv = jnp.pad(visual.transpose(0, 2, 3, 1), ((0, 0), (1, 1), (1, 1), (0, 0)))
    xv = xv.reshape(B, VB, 9, 9, 512)
    Wvt = jnp.transpose(Wv.reshape(512, 512, 9), (2, 1, 0))
    sc_v = (gv * _BN_S).reshape(1, 512)
    bi_v = bv.reshape(1, 512)

    f32 = jnp.float32
    fa_rows, ea = pl.pallas_call(
        _audio_body,
        grid=(B,),
        in_specs=[
            pl.BlockSpec((1, 800, 512), lambda b: (b, 0, 0)),
            pl.BlockSpec((3, 512, 512), lambda b: (0, 0, 0)),
            pl.BlockSpec((2, 512, 512), lambda b: (0, 0, 0)),
            pl.BlockSpec((3, 512, 512), lambda b: (0, 0, 0)),
            pl.BlockSpec((3, 512), lambda b: (0, 0)),
            pl.BlockSpec((3, 512), lambda b: (0, 0)),
        ],
        out_specs=[
            pl.BlockSpec((1, 100, 512), lambda b: (b, 0, 0)),
            pl.BlockSpec((1, 10, 512), lambda b: (b, 0, 0)),
        ],
        out_shape=[
            jax.ShapeDtypeStruct((B, 100, 512), f32),
            jax.ShapeDtypeStruct((B, 10, 512), f32),
        ],
    )(xa, W1t, W2t, W3t, sc_a, bi_a)

    fv_rows, ev = pl.pallas_call(
        _visual_body,
        grid=(B,),
        in_specs=[
            pl.BlockSpec((1, VB, 9, 9, 512), lambda b: (b, 0, 0, 0, 0)),
            pl.BlockSpec((9, 512, 512), lambda b: (0, 0, 0)),
            pl.BlockSpec((1, 512), lambda b: (0, 0)),
            pl.BlockSpec((1, 512), lambda b: (0, 0)),
        ],
        out_specs=[
            pl.BlockSpec((1, VB, 49, 512), lambda b: (b, 0, 0, 0)),
            pl.BlockSpec((1, VB, 512), lambda b: (b, 0, 0)),
        ],
        out_shape=[
            jax.ShapeDtypeStruct((B, VB, 49, 512), f32),
            jax.ShapeDtypeStruct((B, VB, 512), f32),
        ],
    )(xv, Wvt, sc_v, bi_v)

    embed = jnp.concatenate([ea, ev], axis=-1).reshape(B * FRAMES, 1024)
    D1wt = D1w.T
    D2wt = jnp.zeros((128, 128), f32).at[:, :2].set(D2w.T)
    d1b = D1b.reshape(1, 128)
    d2b = jnp.zeros((1, 128), f32).at[0, :2].set(D2b)

    out = pl.pallas_call(
        _mlp_body,
        grid=(1,),
        in_specs=[
            pl.BlockSpec((B * FRAMES, 1024), lambda i: (0, 0)),
            pl.BlockSpec((1024, 128), lambda i: (0, 0)),
            pl.BlockSpec((1, 128), lambda i: (0, 0)),
            pl.BlockSpec((128, 128), lambda i: (0, 0)),
            pl.BlockSpec((1, 128), lambda i: (0, 0)),
        ],
        out_specs=pl.BlockSpec((B * FRAMES, 128), lambda i: (0, 0)),
        out_shape=jax.ShapeDtypeStruct((B * FRAMES, 128), f32),
    )(embed, D1wt, d1b, D2wt, d2b)

    common = out[:, :2].reshape(B, FRAMES, 2)
    feat_a = fa_rows.reshape(B, 50, 2, 512).transpose(0, 3, 1, 2)
    feat_v = fv_rows.reshape(NV, 7, 7, 512).transpose(0, 3, 1, 2)
    return (common, feat_a, feat_v)


# speedup vs baseline: 1.0292x; 1.0292x over previous
import jax
import jax.numpy as jnp
from jax.experimental import pallas as pl

FRAMES = 10
_BN_S = float(1.0 / (1.0 + 1e-5) ** 0.5)


def _audio_body(x_ref, w1_ref, w2_ref, w3_ref, sc_ref, bi_ref, fa_ref, ea_ref):
    bf = jnp.bfloat16
    x = x_ref[0].astype(bf)
    w1 = w1_ref[...]
    sc = sc_ref[...]
    bi = bi_ref[...]
    f32 = jnp.float32

    y0 = jnp.dot(x, w1[0], preferred_element_type=f32)
    y1 = jnp.dot(x, w1[1], preferred_element_type=f32)
    y2 = jnp.dot(x, w1[2], preferred_element_type=f32)
    z8 = jnp.zeros((8, 512), f32)
    y = y1 + jnp.concatenate([z8, y0[:-8]], 0) + jnp.concatenate([y2[8:], z8], 0)
    y = y * sc[0] + bi[0]
    y = y.reshape(100, 2, 4, 512).max(axis=1)
    y = jnp.maximum(y, 0.0)

    p = y.reshape(100, 2, 2, 512).astype(bf)
    ye = p[:, :, 0, :].reshape(200, 512)
    yo = p[:, :, 1, :].reshape(200, 512)
    y = (jnp.dot(ye, w2_ref[0], preferred_element_type=f32)
         + jnp.dot(yo, w2_ref[1], preferred_element_type=f32))
    y = y * sc[1] + bi[1]
    y = jnp.maximum(y, 0.0).astype(bf)

    w3 = w3_ref[...]
    y0 = jnp.dot(y, w3[0], preferred_element_type=f32)
    y1 = jnp.dot(y, w3[1], preferred_element_type=f32)
    y2 = jnp.dot(y, w3[2], preferred_element_type=f32)
    z2 = jnp.zeros((2, 512), f32)
    y = y1 + jnp.concatenate([z2, y0[:-2]], 0) + jnp.concatenate([y2[2:], z2], 0)
    y = y * sc[2] + bi[2]
    y = y.reshape(50, 2, 2, 512).max(axis=1)
    y = jnp.maximum(y, 0.0)

    fa_ref[0] = y.reshape(100, 512)
    ea_ref[0] = y.reshape(10, 5 * 2, 512).max(axis=1)


def _visual_body(x_ref, wv_ref, sc_ref, bi_ref, fv_ref, ev_ref):
    xp = x_ref[0].astype(jnp.bfloat16)
    wv = wv_ref[...]
    f32 = jnp.float32
    acc = jnp.zeros((490, 512), f32)
    k = 0
    for dy in range(3):
        for dx in range(3):
            t = xp[:, dy:dy + 7, dx:dx + 7, :].reshape(490, 512)
            acc = acc + jnp.dot(t, wv[k], preferred_element_type=f32)
            k += 1
    y = acc * sc_ref[...] + bi_ref[...]
    y = jnp.maximum(y, 0.0)
    yb = y.reshape(10, 49, 512)
    fv_ref[0] = yb
    ev_ref[0] = yb.max(axis=1)


def _mlp_body(e_ref, w1_ref, b1_ref, w2_ref, b2_ref, o_ref):
    f32 = jnp.float32
    h = jnp.dot(e_ref[...].astype(jnp.bfloat16), w1_ref[...],
                preferred_element_type=f32) + b1_ref[...]
    h = jnp.maximum(h, 0.0).astype(jnp.bfloat16)
    o_ref[...] = jnp.dot(h, w2_ref[...], preferred_element_type=f32) + b2_ref[...]


def kernel(audio, visual, W1, g1, b1, W2, g2, b2, W3, g3, b3, Wv, gv, bv, D1w, D1b, D2w, D2b):
    B = audio.shape[0]
    NV = visual.shape[0]
    VB = NV // B

    bf = jnp.bfloat16
    xa = audio.transpose(0, 2, 3, 1).reshape(B, 800, 512)
    W1t = jnp.transpose(W1[:, :, :, 0], (2, 1, 0)).astype(bf)
    W2t = jnp.transpose(W2[:, :, 0, :], (2, 1, 0)).astype(bf)
    W3t = jnp.transpose(W3[:, :, :, 0], (2, 1, 0)).astype(bf)
    sc_a = jnp.stack([g1, g2, g3]) * _BN_S
    bi_a = jnp.stack([b1, b2, b3])

    xv = jnp.pad(visual.transpose(0, 2, 3, 1), ((0, 0), (1, 1), (1, 1), (0, 0)))
    xv = xv.reshape(B, VB, 9, 9, 512)
    Wvt = jnp.transpose(Wv.reshape(512, 512, 9), (2, 1, 0)).astype(bf)
    sc_v = (gv * _BN_S).reshape(1, 512)
    bi_v = bv.reshape(1, 512)

    f32 = jnp.float32
    fa_rows, ea = pl.pallas_call(
        _audio_body,
        grid=(B,),
        in_specs=[
            pl.BlockSpec((1, 800, 512), lambda b: (b, 0, 0)),
            pl.BlockSpec((3, 512, 512), lambda b: (0, 0, 0)),
            pl.BlockSpec((2, 512, 512), lambda b: (0, 0, 0)),
            pl.BlockSpec((3, 512, 512), lambda b: (0, 0, 0)),
            pl.BlockSpec((3, 512), lambda b: (0, 0)),
            pl.BlockSpec((3, 512), lambda b: (0, 0)),
        ],
        out_specs=[
            pl.BlockSpec((1, 100, 512), lambda b: (b, 0, 0)),
            pl.BlockSpec((1, 10, 512), lambda b: (b, 0, 0)),
        ],
        out_shape=[
            jax.ShapeDtypeStruct((B, 100, 512), f32),
            jax.ShapeDtypeStruct((B, 10, 512), f32),
        ],
    )(xa, W1t, W2t, W3t, sc_a, bi_a)

    fv_rows, ev = pl.pallas_call(
        _visual_body,
        grid=(B,),
        in_specs=[
            pl.BlockSpec((1, VB, 9, 9, 512), lambda b: (b, 0, 0, 0, 0)),
            pl.BlockSpec((9, 512, 512), lambda b: (0, 0, 0)),
            pl.BlockSpec((1, 512), lambda b: (0, 0)),
            pl.BlockSpec((1, 512), lambda b: (0, 0)),
        ],
        out_specs=[
            pl.BlockSpec((1, VB, 49, 512), lambda b: (b, 0, 0, 0)),
            pl.BlockSpec((1, VB, 512), lambda b: (b, 0, 0)),
        ],
        out_shape=[
            jax.ShapeDtypeStruct((B, VB, 49, 512), f32),
            jax.ShapeDtypeStruct((B, VB, 512), f32),
        ],
    )(xv, Wvt, sc_v, bi_v)

    embed = jnp.concatenate([ea, ev], axis=-1).reshape(B * FRAMES, 1024)
    D1wt = D1w.T.astype(bf)
    D2wt = jnp.zeros((128, 128), bf).at[:, :2].set(D2w.T.astype(bf))
    d1b = D1b.reshape(1, 128)
    d2b = jnp.zeros((1, 128), f32).at[0, :2].set(D2b)

    out = pl.pallas_call(
        _mlp_body,
        grid=(1,),
        in_specs=[
            pl.BlockSpec((B * FRAMES, 1024), lambda i: (0, 0)),
            pl.BlockSpec((1024, 128), lambda i: (0, 0)),
            pl.BlockSpec((1, 128), lambda i: (0, 0)),
            pl.BlockSpec((128, 128), lambda i: (0, 0)),
            pl.BlockSpec((1, 128), lambda i: (0, 0)),
        ],
        out_specs=pl.BlockSpec((B * FRAMES, 128), lambda i: (0, 0)),
        out_shape=jax.ShapeDtypeStruct((B * FRAMES, 128), f32),
    )(embed, D1wt, d1b, D2wt, d2b)

    common = out[:, :2].reshape(B, FRAMES, 2)
    feat_a = fa_rows.reshape(B, 50, 2, 512).transpose(0, 3, 1, 2)
    feat_v = fv_rows.reshape(NV, 7, 7, 512).transpose(0, 3, 1, 2)
    return (common, feat_a, feat_v)
